# baseline (speedup 1.0000x reference)
import functools

import jax
import jax.numpy as jnp
from jax import lax
from jax.experimental import pallas as pl
from jax.experimental.pallas import tpu as pltpu

TEMPERATURE = 0.5
SAMPLE_RATIO = 0.1
L2_WEIGHT = 0.1

N_SAMPLES = 5017
NP = 5120
BR = 256


def _contrastive_body(sp_ref, tp_ref, lab_ref, bat_ref, wgt_ref, out_ref):
    r = pl.program_id(0)
    base = r * BR
    s_blk = sp_ref[pl.ds(base, BR), :]
    logits = lax.dot_general(s_blk, tp_ref[...],
                             (((1,), (1,)), ((), ())),
                             preferred_element_type=jnp.float32)
    logits = logits * (1.0 / TEMPERATURE)

    row_ids = base + lax.broadcasted_iota(jnp.int32, (BR, 1), 0)
    col_ids = lax.broadcasted_iota(jnp.int32, (1, NP), 1)

    lab_r = lab_ref[0, pl.ds(base, BR)].reshape(BR, 1)
    bat_r = bat_ref[0, pl.ds(base, BR)].reshape(BR, 1)
    lab_c = lab_ref[0, :].reshape(1, NP)
    bat_c = bat_ref[0, :].reshape(1, NP)

    valid_col = col_ids < N_SAMPLES
    neg_mask = ((bat_r != bat_c) | (lab_r != lab_c)) & valid_col

    pos = jnp.sum(jnp.where(col_ids == row_ids, logits, 0.0), axis=1,
                  keepdims=True)
    esum = jnp.sum(jnp.where(neg_mask, jnp.exp(logits), 0.0), axis=1,
                   keepdims=True)
    log_prob = pos - jnp.log(jnp.exp(pos) + esum)

    w_r = wgt_ref[0, pl.ds(base, BR)].reshape(BR, 1)
    valid_row = row_ids < N_SAMPLES
    contrib = jnp.sum(jnp.where(valid_row, log_prob * w_r, 0.0))

    @pl.when(r == 0)
    def _():
        out_ref[...] = jnp.zeros((1, 1), jnp.float32)
    out_ref[...] += contrib.reshape(1, 1)


def _contrastive_sum(sp, tp, labels, batch_idx, weights):
    return pl.pallas_call(
        _contrastive_body,
        out_shape=jax.ShapeDtypeStruct((1, 1), jnp.float32),
        grid=(NP // BR,),
        in_specs=[
            pl.BlockSpec((NP, 192), lambda r: (0, 0)),
            pl.BlockSpec((NP, 192), lambda r: (0, 0)),
            pl.BlockSpec((1, NP), lambda r: (0, 0)),
            pl.BlockSpec((1, NP), lambda r: (0, 0)),
            pl.BlockSpec((1, NP), lambda r: (0, 0)),
        ],
        out_specs=pl.BlockSpec((1, 1), lambda r: (0, 0)),
    )(sp, tp, labels, batch_idx, weights)


def kernel(student_feat, teacher_feat, teacher_logits, conv1_w, conv2_w,
           conv2_b):
    dn = ('NCHW', 'OIHW', 'NCHW')
    B, C, H, W = student_feat.shape

    probs = jax.nn.softmax(teacher_logits, axis=1)
    max_probs = jnp.max(probs, axis=1, keepdims=True)
    h = lax.conv_general_dilated(max_probs, conv1_w, (1, 1), 'SAME',
                                 dimension_numbers=dn)
    h = jax.nn.relu(h)
    h = lax.conv_general_dilated(h, conv2_w, (1, 1), 'SAME',
                                 dimension_numbers=dn)
    h = h + conv2_b.reshape(1, -1, 1, 1)
    boundary = jax.nn.sigmoid(h)
    boundary = boundary / (jnp.max(boundary) + 1e-06)

    s_norm = jnp.maximum(
        jnp.sqrt(jnp.sum(student_feat ** 2, axis=1, keepdims=True)), 1e-12)
    t_norm = jnp.maximum(
        jnp.sqrt(jnp.sum(teacher_feat ** 2, axis=1, keepdims=True)), 1e-12)
    s_feat = student_feat / s_norm
    t_feat = teacher_feat / t_norm

    sample_weights = boundary.reshape(-1) + 1e-06
    sample_weights = sample_weights / jnp.sum(sample_weights)
    skey = jax.random.key(42)
    indices = jax.random.categorical(skey, jnp.log(sample_weights),
                                     shape=(N_SAMPLES,))

    s_pixels = jnp.transpose(s_feat, (0, 2, 3, 1)).reshape(-1, C)[indices]
    t_pixels = jnp.transpose(t_feat, (0, 2, 3, 1)).reshape(-1, C)[indices]
    batch_indices = jnp.repeat(jnp.arange(B), H * W)[indices]
    labels = jnp.argmax(teacher_logits, axis=1).reshape(-1)[indices]
    boundary_weights = boundary.reshape(-1)[indices].astype(jnp.float32)
    weights = 1.0 + boundary_weights

    pad = NP - N_SAMPLES
    sp = jnp.pad(s_pixels, ((0, pad), (0, 0)))
    tp = jnp.pad(t_pixels, ((0, pad), (0, 0)))
    lab = jnp.pad(labels.astype(jnp.int32), (0, pad)).reshape(1, NP)
    bat = jnp.pad(batch_indices.astype(jnp.int32), (0, pad),
                  constant_values=-1).reshape(1, NP)
    wgt = jnp.pad(weights, (0, pad)).reshape(1, NP)

    total = _contrastive_sum(sp, tp, lab, bat, wgt)[0, 0]
    contrastive_loss = -total / N_SAMPLES

    l2_loss = jnp.mean((s_feat - t_feat) ** 2)
    return contrastive_loss + L2_WEIGHT * l2_loss


# speedup vs baseline: 9.7664x; 9.7664x over previous
import functools

import jax
import jax.numpy as jnp
import numpy as np
from jax import lax
from jax.experimental import pallas as pl
from jax.experimental.pallas import tpu as pltpu
from jax.experimental.pallas import tpu_sc as plsc

TEMPERATURE = 0.5
L2_WEIGHT = 0.1

B, C, H, W = 2, 192, 224, 224
HW = H * W
NPIX = B * HW
N_SAMPLES = 5017
NP = 5120
BR = 256
NROW = NPIX // 128
SG = 256
NG = NP // SG

_MASK32 = 0xFFFFFFFF


def _softmax_body(tl_ref, mp_ref, lab_ref):
    x0 = tl_ref[0, 0]
    mx = x0
    amx = jnp.zeros_like(x0, dtype=jnp.int32)
    for c in range(1, 21):
        xc = tl_ref[0, c]
        upd = xc > mx
        mx = jnp.where(upd, xc, mx)
        amx = jnp.where(upd, c, amx)
    den = jnp.zeros_like(x0)
    for c in range(21):
        den = den + jnp.exp(tl_ref[0, c] - mx)
    mp_ref[0] = 1.0 / den
    lab_ref[0] = amx.astype(jnp.float32)


def _softmax_stage(teacher_logits):
    return pl.pallas_call(
        _softmax_body,
        out_shape=(
            jax.ShapeDtypeStruct((B, H, W), jnp.float32),
            jax.ShapeDtypeStruct((B, H, W), jnp.float32),
        ),
        grid=(B,),
        in_specs=[pl.BlockSpec((1, 21, H, W), lambda b: (b, 0, 0, 0))],
        out_specs=(
            pl.BlockSpec((1, H, W), lambda b: (b, 0, 0)),
            pl.BlockSpec((1, H, W), lambda b: (b, 0, 0)),
        ),
    )(teacher_logits)


def _conv_body(mp_ref, w1_ref, w2_ref, b2_ref, raw_ref, bmax_ref):
    acc = jnp.full((H, W), b2_ref[0, 0], jnp.float32)
    for k in range(16):
        hk = jnp.zeros((H, W), jnp.float32)
        for dy in range(3):
            for dx in range(3):
                hk = hk + w1_ref[k, 3 * dy + dx] * mp_ref[0, dy:dy + H,
                                                          dx:dx + W]
        acc = acc + w2_ref[0, k] * jnp.maximum(hk, 0.0)
    raw = 1.0 / (1.0 + jnp.exp(-acc))
    raw_ref[0] = raw
    bmax_ref[0] = jnp.max(raw).reshape(1, 1)


def _conv_stage(mp_pad, w1, w2, b2):
    return pl.pallas_call(
        _conv_body,
        out_shape=(
            jax.ShapeDtypeStruct((B, H, W), jnp.float32),
            jax.ShapeDtypeStruct((B, 1, 1), jnp.float32),
        ),
        grid=(B,),
        in_specs=[
            pl.BlockSpec((1, H + 2, W + 2), lambda b: (b, 0, 0)),
            pl.BlockSpec(memory_space=pltpu.SMEM),
            pl.BlockSpec(memory_space=pltpu.SMEM),
            pl.BlockSpec(memory_space=pltpu.SMEM),
        ],
        out_specs=(
            pl.BlockSpec((1, H, W), lambda b: (b, 0, 0)),
            pl.BlockSpec((1, 1, 1), lambda b: (b, 0, 0)),
        ),
    )(mp_pad, w1, w2, b2)


def _threefry_bits(cnt):
    ks0 = jnp.int32(0)
    ks1 = jnp.int32(42)
    ks2 = jnp.int32((0 ^ 42 ^ 0x1BD11BDA) & _MASK32)
    ks = (ks0, ks1, ks2)
    rot0 = (13, 15, 26, 6)
    rot1 = (17, 29, 16, 24)
    x1 = jnp.zeros_like(cnt) + ks0
    x2 = cnt + ks1
    for i, rots in enumerate((rot0, rot1, rot0, rot1, rot0)):
        for r in rots:
            x1 = x1 + x2
            x2 = (lax.shift_left(x2, jnp.int32(r))
                  | lax.shift_right_logical(x2, jnp.int32(32 - r)))
            x2 = lax.bitwise_xor(x2, x1)
        x1 = x1 + ks[(i + 1) % 3]
        x2 = x2 + ks[(i + 2) % 3] + jnp.int32(i + 1)
    return lax.bitwise_xor(x1, x2)


def _bits_to_unit(bits):
    f = lax.bitcast_convert_type(
        lax.shift_right_logical(bits, jnp.int32(9)) | jnp.int32(0x3F800000),
        jnp.float32)
    return f - 1.0


def _sampler_body(wraw_ref, labf_ref, m_ref, g_ref, lab_ref, bat_ref,
                  wgt_ref):
    m = m_ref[0]
    wv = wraw_ref[...] / (m + 1e-06) + 1e-06
    labf = labf_ref[...]

    li = lax.broadcasted_iota(jnp.int32, (128, 128), 0)
    lj = lax.broadcasted_iota(jnp.int32, (128, 128), 1)
    tri = (li <= lj).astype(jnp.float32)
    lcum = lax.dot_general(wv, tri, (((1,), (0,)), ((), ())),
                           preferred_element_type=jnp.float32)

    ones_col = jnp.ones((128, 1), jnp.float32)
    trow = lax.dot_general(wv, ones_col, (((1,), (0,)), ((), ())),
                           preferred_element_type=jnp.float32)

    ri = lax.broadcasted_iota(jnp.int32, (NROW, NROW), 0)
    rj = lax.broadcasted_iota(jnp.int32, (NROW, NROW), 1)
    lowtri = (rj <= ri).astype(jnp.float32)
    p_incl = lax.dot_general(lowtri, trow, (((1,), (0,)), ((), ())),
                             preferred_element_type=jnp.float32)
    p_excl = p_incl - trow
    ident = (ri == rj).astype(jnp.float32)
    p_row = lax.dot_general(p_incl, ident, (((0,), (0,)), ((), ())),
                            preferred_element_type=jnp.float32)
    total = jnp.max(p_incl)

    cnt = (lax.broadcasted_iota(jnp.int32, (SG, NG), 0)
           + SG * lax.broadcasted_iota(jnp.int32, (SG, NG), 1))
    u = _bits_to_unit(_threefry_bits(cnt))
    tthr = u * total

    lane = lax.broadcasted_iota(jnp.int32, (1, 128), 1)
    for g in range(NG):
        t_g = tthr[:, g:g + 1]
        cmp = (p_row <= t_g).astype(jnp.float32)
        b = jnp.sum(cmp, axis=1, keepdims=True)
        b = jnp.minimum(b, float(NROW - 1))
        bi = b.astype(jnp.int32)
        rid = lax.broadcasted_iota(jnp.int32, (SG, NROW), 1)
        onehot = (rid == bi).astype(jnp.float32)
        rows = lax.dot_general(onehot, lcum, (((1,), (0,)), ((), ())),
                               preferred_element_type=jnp.float32)
        offs = lax.dot_general(onehot, p_excl, (((1,), (0,)), ((), ())),
                               preferred_element_type=jnp.float32)
        wrow = lax.dot_general(onehot, wv, (((1,), (0,)), ((), ())),
                               preferred_element_type=jnp.float32)
        lrow = lax.dot_general(onehot, labf, (((1,), (0,)), ((), ())),
                               preferred_element_type=jnp.float32)
        fine = jnp.sum((offs + rows <= t_g).astype(jnp.float32), axis=1,
                       keepdims=True)
        fine = jnp.minimum(fine, 127.0).astype(jnp.int32)
        lsel = (lane == fine).astype(jnp.float32)
        w_k = jnp.sum(lsel * wrow, axis=1, keepdims=True)
        l_k = jnp.sum(lsel * lrow, axis=1, keepdims=True)
        gidx = bi * 128 + fine
        g_ref[:, g:g + 1] = gidx
        lab_ref[:, g:g + 1] = (l_k + 0.5).astype(jnp.int32)
        bat_ref[:, g:g + 1] = (gidx >= HW).astype(jnp.int32)
        wgt_ref[:, g:g + 1] = w_k + (1.0 - 1e-06)


def _sampler_stage(wraw, labf, m):
    return pl.pallas_call(
        _sampler_body,
        out_shape=(
            jax.ShapeDtypeStruct((SG, NG), jnp.int32),
            jax.ShapeDtypeStruct((SG, NG), jnp.int32),
            jax.ShapeDtypeStruct((SG, NG), jnp.int32),
            jax.ShapeDtypeStruct((SG, NG), jnp.float32),
        ),
        in_specs=[
            pl.BlockSpec((NROW, 128), lambda: (0, 0)),
            pl.BlockSpec((NROW, 128), lambda: (0, 0)),
            pl.BlockSpec(memory_space=pltpu.SMEM),
        ],
        out_specs=(
            pl.BlockSpec((SG, NG), lambda: (0, 0)),
            pl.BlockSpec((SG, NG), lambda: (0, 0)),
            pl.BlockSpec((SG, NG), lambda: (0, 0)),
            pl.BlockSpec((SG, NG), lambda: (0, 0)),
        ),
    )(wraw, labf, m)


RL2 = 16


def _l2_body(s_ref, t_ref, out_ref):
    s = s_ref[0]
    t = t_ref[0]
    ss = jnp.sum(s * s, axis=0)
    st = jnp.sum(s * t, axis=0)
    tt = jnp.sum(t * t, axis=0)
    denom = (jnp.maximum(jnp.sqrt(ss), 1e-12)
             * jnp.maximum(jnp.sqrt(tt), 1e-12))
    part = jnp.sum(1.0 - st / denom)

    i = pl.program_id(0)
    j = pl.program_id(1)

    @pl.when((i == 0) & (j == 0))
    def _():
        out_ref[...] = jnp.zeros((1, 1), jnp.float32)
    out_ref[...] += part.reshape(1, 1)


def _l2_stage(student_feat, teacher_feat):
    return pl.pallas_call(
        _l2_body,
        out_shape=jax.ShapeDtypeStruct((1, 1), jnp.float32),
        grid=(B, H // RL2),
        in_specs=[
            pl.BlockSpec((1, C, RL2, W), lambda b, r: (b, 0, r, 0)),
            pl.BlockSpec((1, C, RL2, W), lambda b, r: (b, 0, r, 0)),
        ],
        out_specs=pl.BlockSpec((1, 1), lambda b, r: (0, 0)),
    )(student_feat, teacher_feat)


NW = 32
SPW = NP // NW
SGRP = 16
CROWS = HW // 16
NVROW = B * C * CROWS
NCHUNK = C * SGRP // 128


def _sc_gather_kernel(s_hbm, t_hbm, g_hbm, s_out, t_out,
                      g_v, idx_v, rows_s, rows_t, out_s, out_t,
                      sem_s, sem_t):
    wid = lax.axis_index("s") * 2 + lax.axis_index("c")
    base = wid * SPW
    pltpu.sync_copy(g_hbm.at[pl.ds(base, SPW)], g_v)
    lane16 = lax.iota(jnp.int32, 16)

    def group_body(gi, _):
        gofs = gi * SGRP
        g16 = g_v[pl.ds(gofs, SGRP)]
        bvec = (g16 >= HW).astype(jnp.int32)
        ovec = g16 - bvec * HW
        lanev = lax.bitwise_and(ovec, jnp.int32(15))
        basev = bvec * (C * CROWS) + lax.shift_right_logical(ovec,
                                                             jnp.int32(4))

        def build_idx(c, _):
            idx_v[lax.div(c, jnp.int32(8)),
                  pl.ds(lax.rem(c, jnp.int32(8)) * SGRP, SGRP)] = (
                      basev + c * CROWS)
            return ()
        lax.fori_loop(0, C, build_idx, (), unroll=8)

        cps = []
        for j in range(NCHUNK):
            cps.append(pltpu.async_copy(
                s_hbm.at[idx_v.at[j]], rows_s.at[pl.ds(j * 128, 128)],
                sem_s))
            cps.append(pltpu.async_copy(
                t_hbm.at[idx_v.at[j]], rows_t.at[pl.ds(j * 128, 128)],
                sem_t))
        for cp in cps:
            cp.wait()

        def extract(c, _):
            rowsel = c * SGRP + lane16
            cvec = jnp.full((SGRP,), 0, jnp.int32) + c
            vs = plsc.load_gather(rows_s, [rowsel, lanev])
            vt = plsc.load_gather(rows_t, [rowsel, lanev])
            plsc.store_scatter(out_s, [gofs + lane16, cvec], vs)
            plsc.store_scatter(out_t, [gofs + lane16, cvec], vt)
            return ()
        lax.fori_loop(0, C, extract, (), unroll=4)
        return ()

    lax.fori_loop(0, SPW // SGRP, group_body, ())
    pltpu.sync_copy(out_s, s_out.at[pl.ds(base, SPW)])
    pltpu.sync_copy(out_t, t_out.at[pl.ds(base, SPW)])


def _sc_gather(s2d, t2d, gidx):
    mesh = plsc.VectorSubcoreMesh(core_axis_name="c", subcore_axis_name="s")
    return pl.kernel(
        _sc_gather_kernel,
        out_type=(
            jax.ShapeDtypeStruct((NP, C), jnp.float32),
            jax.ShapeDtypeStruct((NP, C), jnp.float32),
        ),
        mesh=mesh,
        scratch_types=[
            pltpu.VMEM((SPW,), jnp.int32),
            pltpu.VMEM((NCHUNK, 128), jnp.int32),
            pltpu.VMEM((C * SGRP, 16), jnp.float32),
            pltpu.VMEM((C * SGRP, 16), jnp.float32),
            pltpu.VMEM((SPW, C), jnp.float32),
            pltpu.VMEM((SPW, C), jnp.float32),
            pltpu.SemaphoreType.DMA,
            pltpu.SemaphoreType.DMA,
        ],
    )(s2d, t2d, gidx)


def _contrastive_body(sp_ref, tp_ref, lab_ref, bat_ref, wgt_ref, out_ref):
    r = pl.program_id(0)
    base = r * BR

    tp = tp_ref[...]
    tt = jnp.sum(tp * tp, axis=1, keepdims=True)
    inv_t = 1.0 / jnp.maximum(jnp.sqrt(tt), 1e-12)

    s_blk = sp_ref[pl.ds(base, BR), :]
    ssq = jnp.sum(s_blk * s_blk, axis=1, keepdims=True)
    inv_s = 1.0 / jnp.maximum(jnp.sqrt(ssq), 1e-12)

    dots = lax.dot_general(s_blk, tp, (((1,), (1,)), ((), ())),
                           preferred_element_type=jnp.float32)
    logits = dots * inv_s * inv_t.reshape(1, NP) * (1.0 / TEMPERATURE)

    row_ids = base + lax.broadcasted_iota(jnp.int32, (BR, 1), 0)
    col_ids = lax.broadcasted_iota(jnp.int32, (1, NP), 1)

    lab_r = lab_ref[0, pl.ds(base, BR)].reshape(BR, 1)
    bat_r = bat_ref[0, pl.ds(base, BR)].reshape(BR, 1)
    lab_c = lab_ref[0, :].reshape(1, NP)
    bat_c = bat_ref[0, :].reshape(1, NP)

    valid_col = col_ids < N_SAMPLES
    neg_mask = ((bat_r != bat_c) | (lab_r != lab_c)) & valid_col

    pos = jnp.sum(jnp.where(col_ids == row_ids, logits, 0.0), axis=1,
                  keepdims=True)
    esum = jnp.sum(jnp.where(neg_mask, jnp.exp(logits), 0.0), axis=1,
                   keepdims=True)
    log_prob = pos - jnp.log(jnp.exp(pos) + esum)

    w_r = wgt_ref[0, pl.ds(base, BR)].reshape(BR, 1)
    valid_row = row_ids < N_SAMPLES
    contrib = jnp.sum(jnp.where(valid_row, log_prob * w_r, 0.0))

    @pl.when(r == 0)
    def _():
        out_ref[...] = jnp.zeros((1, 1), jnp.float32)
    out_ref[...] += contrib.reshape(1, 1)


def _contrastive_sum(sp, tp, labels, batch_idx, weights):
    return pl.pallas_call(
        _contrastive_body,
        out_shape=jax.ShapeDtypeStruct((1, 1), jnp.float32),
        grid=(NP // BR,),
        in_specs=[
            pl.BlockSpec((NP, C), lambda r: (0, 0)),
            pl.BlockSpec((NP, C), lambda r: (0, 0)),
            pl.BlockSpec((1, NP), lambda r: (0, 0)),
            pl.BlockSpec((1, NP), lambda r: (0, 0)),
            pl.BlockSpec((1, NP), lambda r: (0, 0)),
        ],
        out_specs=pl.BlockSpec((1, 1), lambda r: (0, 0)),
    )(sp, tp, labels, batch_idx, weights)


def kernel(student_feat, teacher_feat, teacher_logits, conv1_w, conv2_w,
           conv2_b):
    mp, labf = _softmax_stage(teacher_logits)
    mp_pad = jnp.pad(mp, ((0, 0), (1, 1), (1, 1)))
    w1 = conv1_w.reshape(16, 9)
    w2 = conv2_w.reshape(1, 16)
    b2 = conv2_b.reshape(1, 1)
    raw, bmax = _conv_stage(mp_pad, w1, w2, b2)

    m = jnp.max(bmax).reshape(1)
    wraw = raw.reshape(NROW, 128)
    labf2d = labf.reshape(NROW, 128)

    g2d, lab2d, bat2d, wgt2d = _sampler_stage(wraw, labf2d, m)
    gidx = g2d.T.reshape(NP)
    labels = lab2d.T.reshape(NP)
    batch_idx = bat2d.T.reshape(NP)
    weights = wgt2d.T.reshape(NP)

    s_pix = jnp.transpose(student_feat, (0, 2, 3, 1)).reshape(-1, C)[gidx]
    t_pix = jnp.transpose(teacher_feat, (0, 2, 3, 1)).reshape(-1, C)[gidx]

    total = _contrastive_sum(s_pix, t_pix,
                             labels.reshape(1, NP), batch_idx.reshape(1, NP),
                             weights.reshape(1, NP))[0, 0]
    contrastive_loss = -total / N_SAMPLES

    cos_sum = _l2_stage(student_feat, teacher_feat)[0, 0]
    l2_loss = (2.0 / C) * (cos_sum / NPIX)
    return contrastive_loss + L2_WEIGHT * l2_loss


# speedup vs baseline: 22.7604x; 2.3305x over previous
import functools

import jax
import jax.numpy as jnp
import numpy as np
from jax import lax
from jax.experimental import pallas as pl
from jax.experimental.pallas import tpu as pltpu
from jax.experimental.pallas import tpu_sc as plsc

TEMPERATURE = 0.5
L2_WEIGHT = 0.1

B, C, H, W = 2, 192, 224, 224
HW = H * W
NPIX = B * HW
N_SAMPLES = 5017
NP = 5120
BR = 512
NROW = NPIX // 128
SG = 256
NG = NP // SG

_MASK32 = 0xFFFFFFFF


def _softmax_body(tl_ref, mp_ref, lab_ref):
    x0 = tl_ref[0, 0]
    mx = x0
    amx = jnp.zeros_like(x0, dtype=jnp.int32)
    for c in range(1, 21):
        xc = tl_ref[0, c]
        upd = xc > mx
        mx = jnp.where(upd, xc, mx)
        amx = jnp.where(upd, c, amx)
    den = jnp.zeros_like(x0)
    for c in range(21):
        den = den + jnp.exp(tl_ref[0, c] - mx)
    mp_ref[0] = 1.0 / den
    lab_ref[0] = amx.astype(jnp.float32)


def _softmax_stage(teacher_logits):
    return pl.pallas_call(
        _softmax_body,
        out_shape=(
            jax.ShapeDtypeStruct((B, H, W), jnp.float32),
            jax.ShapeDtypeStruct((B, H, W), jnp.float32),
        ),
        grid=(B,),
        in_specs=[pl.BlockSpec((1, 21, H, W), lambda b: (b, 0, 0, 0))],
        out_specs=(
            pl.BlockSpec((1, H, W), lambda b: (b, 0, 0)),
            pl.BlockSpec((1, H, W), lambda b: (b, 0, 0)),
        ),
    )(teacher_logits)


def _conv_body(mp_ref, w1_ref, w2_ref, b2_ref, raw_ref, bmax_ref):
    acc = jnp.full((H, W), b2_ref[0, 0], jnp.float32)
    for k in range(16):
        hk = jnp.zeros((H, W), jnp.float32)
        for dy in range(3):
            for dx in range(3):
                hk = hk + w1_ref[k, 3 * dy + dx] * mp_ref[0, dy:dy + H,
                                                          dx:dx + W]
        acc = acc + w2_ref[0, k] * jnp.maximum(hk, 0.0)
    raw = 1.0 / (1.0 + jnp.exp(-acc))
    raw_ref[0] = raw
    bmax_ref[0] = jnp.max(raw).reshape(1, 1)


def _conv_stage(mp_pad, w1, w2, b2):
    return pl.pallas_call(
        _conv_body,
        out_shape=(
            jax.ShapeDtypeStruct((B, H, W), jnp.float32),
            jax.ShapeDtypeStruct((B, 1, 1), jnp.float32),
        ),
        grid=(B,),
        in_specs=[
            pl.BlockSpec((1, H + 2, W + 2), lambda b: (b, 0, 0)),
            pl.BlockSpec(memory_space=pltpu.SMEM),
            pl.BlockSpec(memory_space=pltpu.SMEM),
            pl.BlockSpec(memory_space=pltpu.SMEM),
        ],
        out_specs=(
            pl.BlockSpec((1, H, W), lambda b: (b, 0, 0)),
            pl.BlockSpec((1, 1, 1), lambda b: (b, 0, 0)),
        ),
    )(mp_pad, w1, w2, b2)


def _threefry_bits(cnt):
    ks0 = jnp.int32(0)
    ks1 = jnp.int32(42)
    ks2 = jnp.int32((0 ^ 42 ^ 0x1BD11BDA) & _MASK32)
    ks = (ks0, ks1, ks2)
    rot0 = (13, 15, 26, 6)
    rot1 = (17, 29, 16, 24)
    x1 = jnp.zeros_like(cnt) + ks0
    x2 = cnt + ks1
    for i, rots in enumerate((rot0, rot1, rot0, rot1, rot0)):
        for r in rots:
            x1 = x1 + x2
            x2 = (lax.shift_left(x2, jnp.int32(r))
                  | lax.shift_right_logical(x2, jnp.int32(32 - r)))
            x2 = lax.bitwise_xor(x2, x1)
        x1 = x1 + ks[(i + 1) % 3]
        x2 = x2 + ks[(i + 2) % 3] + jnp.int32(i + 1)
    return lax.bitwise_xor(x1, x2)


def _bits_to_unit(bits):
    f = lax.bitcast_convert_type(
        lax.shift_right_logical(bits, jnp.int32(9)) | jnp.int32(0x3F800000),
        jnp.float32)
    return f - 1.0


def _sampler_body(wraw_ref, labf_ref, m_ref, g_ref, key_ref, wgt_ref):
    m = m_ref[0]
    wv = wraw_ref[...] / (m + 1e-06) + 1e-06
    labf = labf_ref[...]

    li = lax.broadcasted_iota(jnp.int32, (128, 128), 0)
    lj = lax.broadcasted_iota(jnp.int32, (128, 128), 1)
    tri = (li <= lj).astype(jnp.float32)
    lcum = lax.dot_general(wv, tri, (((1,), (0,)), ((), ())),
                           preferred_element_type=jnp.float32)

    ones_col = jnp.ones((128, 1), jnp.float32)
    trow = lax.dot_general(wv, ones_col, (((1,), (0,)), ((), ())),
                           preferred_element_type=jnp.float32)

    ri = lax.broadcasted_iota(jnp.int32, (NROW, NROW), 0)
    rj = lax.broadcasted_iota(jnp.int32, (NROW, NROW), 1)
    lowtri = (rj <= ri).astype(jnp.float32)
    p_incl = lax.dot_general(lowtri, trow, (((1,), (0,)), ((), ())),
                             preferred_element_type=jnp.float32)
    p_excl = p_incl - trow
    ident = (ri == rj).astype(jnp.float32)
    p_row = lax.dot_general(p_incl, ident, (((0,), (0,)), ((), ())),
                            preferred_element_type=jnp.float32)
    total = jnp.max(p_incl)

    cnt = (lax.broadcasted_iota(jnp.int32, (SG, NG), 0)
           + SG * lax.broadcasted_iota(jnp.int32, (SG, NG), 1))
    u = _bits_to_unit(_threefry_bits(cnt))
    tthr = u * total

    lane = lax.broadcasted_iota(jnp.int32, (1, 128), 1)
    for g in range(NG):
        t_g = tthr[:, g:g + 1]
        cmp = (p_row <= t_g).astype(jnp.float32)
        b = jnp.sum(cmp, axis=1, keepdims=True)
        b = jnp.minimum(b, float(NROW - 1))
        bi = b.astype(jnp.int32)
        rid = lax.broadcasted_iota(jnp.int32, (SG, NROW), 1)
        onehot = (rid == bi).astype(jnp.float32)
        rows = lax.dot_general(onehot, lcum, (((1,), (0,)), ((), ())),
                               preferred_element_type=jnp.float32)
        offs = lax.dot_general(onehot, p_excl, (((1,), (0,)), ((), ())),
                               preferred_element_type=jnp.float32)
        wrow = lax.dot_general(onehot, wv, (((1,), (0,)), ((), ())),
                               preferred_element_type=jnp.float32)
        lrow = lax.dot_general(onehot, labf, (((1,), (0,)), ((), ())),
                               preferred_element_type=jnp.float32)
        fine = jnp.sum((offs + rows <= t_g).astype(jnp.float32), axis=1,
                       keepdims=True)
        fine = jnp.minimum(fine, 127.0).astype(jnp.int32)
        lsel = (lane == fine).astype(jnp.float32)
        w_k = jnp.sum(lsel * wrow, axis=1, keepdims=True)
        l_k = jnp.sum(lsel * lrow, axis=1, keepdims=True)
        gidx = bi * 128 + fine
        g_ref[:, g:g + 1] = gidx
        lab_i = (l_k + 0.5).astype(jnp.int32)
        key_ref[:, g:g + 1] = jnp.where(gidx >= HW, 32, 0) + lab_i
        wgt_ref[:, g:g + 1] = w_k + (1.0 - 1e-06)


def _sampler_stage(wraw, labf, m):
    return pl.pallas_call(
        _sampler_body,
        out_shape=(
            jax.ShapeDtypeStruct((SG, NG), jnp.int32),
            jax.ShapeDtypeStruct((SG, NG), jnp.int32),
            jax.ShapeDtypeStruct((SG, NG), jnp.float32),
        ),
        in_specs=[
            pl.BlockSpec((NROW, 128), lambda: (0, 0)),
            pl.BlockSpec((NROW, 128), lambda: (0, 0)),
            pl.BlockSpec(memory_space=pltpu.SMEM),
        ],
        out_specs=(
            pl.BlockSpec((SG, NG), lambda: (0, 0)),
            pl.BlockSpec((SG, NG), lambda: (0, 0)),
            pl.BlockSpec((SG, NG), lambda: (0, 0)),
        ),
    )(wraw, labf, m)


RL2 = 32


def _l2_body(s_ref, t_ref, out_ref):
    s = s_ref[0]
    t = t_ref[0]
    ss = jnp.sum(s * s, axis=0)
    st = jnp.sum(s * t, axis=0)
    tt = jnp.sum(t * t, axis=0)
    denom = (jnp.maximum(jnp.sqrt(ss), 1e-12)
             * jnp.maximum(jnp.sqrt(tt), 1e-12))
    part = jnp.sum(1.0 - st / denom)

    i = pl.program_id(0)
    j = pl.program_id(1)

    @pl.when((i == 0) & (j == 0))
    def _():
        out_ref[...] = jnp.zeros((1, 1), jnp.float32)
    out_ref[...] += part.reshape(1, 1)


def _l2_stage(student_feat, teacher_feat):
    return pl.pallas_call(
        _l2_body,
        out_shape=jax.ShapeDtypeStruct((1, 1), jnp.float32),
        grid=(B, H // RL2),
        in_specs=[
            pl.BlockSpec((1, C, RL2, W), lambda b, r: (b, 0, r, 0)),
            pl.BlockSpec((1, C, RL2, W), lambda b, r: (b, 0, r, 0)),
        ],
        out_specs=pl.BlockSpec((1, 1), lambda b, r: (0, 0)),
    )(student_feat, teacher_feat)


NW = 32
SPW = NP // NW
SGRP = 16
CROWS = HW // 16
NVROW = B * C * CROWS
CH = 96
CPB = CH * SGRP // 128
NGRP = SPW // SGRP


def _sc_gather_kernel(s_hbm, t_hbm, g_hbm, s_out, t_out,
                      g_v, idx0, idx1, rs0, rs1, rt0, rt1, out_s, out_t,
                      sem_s0, sem_s1, sem_t0, sem_t1):
    wid = lax.axis_index("s") * 2 + lax.axis_index("c")
    base = wid * SPW
    pltpu.sync_copy(g_hbm.at[pl.ds(base, SPW)], g_v)
    lane16 = lax.iota(jnp.int32, 16)

    def lanev_of(s):
        g16 = g_v[pl.ds((s // 2) * SGRP, SGRP)]
        bvec = jnp.where(g16 >= HW,
                         jnp.full((SGRP,), 1, jnp.int32),
                         jnp.full((SGRP,), 0, jnp.int32))
        ovec = g16 - bvec * HW
        lanev = lax.bitwise_and(ovec, jnp.int32(15))
        basev = bvec * (C * CROWS) + lax.shift_right_logical(ovec,
                                                             jnp.int32(4))
        return lanev, basev

    def build_fire(s, idx_ref, rs_ref, rt_ref, ss, st):
        _, basev = lanev_of(s)
        cbase = lax.rem(s, jnp.int32(2)) * CH

        def bi(c, _):
            idx_ref[pl.ds(c * SGRP, SGRP)] = basev + (cbase + c) * CROWS
            return ()
        lax.fori_loop(0, CH, bi, (), unroll=8)
        for j in range(CPB):
            pltpu.async_copy(s_hbm.at[idx_ref.at[pl.ds(j * 128, 128)]],
                             rs_ref.at[pl.ds(j * 128, 128)], ss)
            pltpu.async_copy(t_hbm.at[idx_ref.at[pl.ds(j * 128, 128)]],
                             rt_ref.at[pl.ds(j * 128, 128)], st)

    def drain(rs_ref, rt_ref, ss, st):
        for j in range(CPB):
            pltpu.make_async_copy(s_hbm.at[pl.ds(0, 128)],
                                  rs_ref.at[pl.ds(j * 128, 128)], ss).wait()
            pltpu.make_async_copy(s_hbm.at[pl.ds(0, 128)],
                                  rt_ref.at[pl.ds(j * 128, 128)], st).wait()

    def extract(s, rs_ref, rt_ref):
        lanev, _ = lanev_of(s)
        cbase = lax.rem(s, jnp.int32(2)) * CH

        def ex(c, _):
            rowsel = c * SGRP + lane16
            cvec = jnp.full((SGRP,), 0, jnp.int32) + cbase + c
            vs = plsc.load_gather(rs_ref, [rowsel, lanev])
            vt = plsc.load_gather(rt_ref, [rowsel, lanev])
            plsc.store_scatter(out_s, [lane16, cvec], vs)
            plsc.store_scatter(out_t, [lane16, cvec], vt)
            return ()
        lax.fori_loop(0, CH, ex, (), unroll=4)

    build_fire(jnp.int32(0), idx0, rs0, rt0, sem_s0, sem_t0)

    def loop(k, _):
        s0 = 2 * k
        s1 = 2 * k + 1
        build_fire(s1, idx1, rs1, rt1, sem_s1, sem_t1)
        drain(rs0, rt0, sem_s0, sem_t0)
        extract(s0, rs0, rt0)

        @pl.when(k < NGRP - 1)
        def _():
            build_fire(s0 + 2, idx0, rs0, rt0, sem_s0, sem_t0)

        drain(rs1, rt1, sem_s1, sem_t1)
        extract(s1, rs1, rt1)
        pltpu.sync_copy(out_s, s_out.at[pl.ds(base + k * SGRP, SGRP)])
        pltpu.sync_copy(out_t, t_out.at[pl.ds(base + k * SGRP, SGRP)])
        return ()

    lax.fori_loop(0, NGRP, loop, ())


def _sc_gather(s2d, t2d, gidx):
    mesh = plsc.VectorSubcoreMesh(core_axis_name="c", subcore_axis_name="s")
    return pl.kernel(
        _sc_gather_kernel,
        out_type=(
            jax.ShapeDtypeStruct((NP, C), jnp.float32),
            jax.ShapeDtypeStruct((NP, C), jnp.float32),
        ),
        mesh=mesh,
        scratch_types=[
            pltpu.VMEM((SPW,), jnp.int32),
            pltpu.VMEM((CH * SGRP,), jnp.int32),
            pltpu.VMEM((CH * SGRP,), jnp.int32),
            pltpu.VMEM((CH * SGRP, 16), jnp.float32),
            pltpu.VMEM((CH * SGRP, 16), jnp.float32),
            pltpu.VMEM((CH * SGRP, 16), jnp.float32),
            pltpu.VMEM((CH * SGRP, 16), jnp.float32),
            pltpu.VMEM((SGRP, C), jnp.float32),
            pltpu.VMEM((SGRP, C), jnp.float32),
            pltpu.SemaphoreType.DMA,
            pltpu.SemaphoreType.DMA,
            pltpu.SemaphoreType.DMA,
            pltpu.SemaphoreType.DMA,
        ],
        compiler_params=pltpu.CompilerParams(
            use_tc_tiling_on_sc=False, needs_layout_passes=False),
    )(s2d, t2d, gidx)


def _contrastive_body(sp_ref, tp_ref, key_ref, wgt_ref, out_ref):
    r = pl.program_id(0)
    base = r * BR

    tp = tp_ref[...]
    tt = jnp.sum(tp * tp, axis=1, keepdims=True)
    inv_t = 1.0 / jnp.maximum(jnp.sqrt(tt), 1e-12)

    s_blk = sp_ref[pl.ds(base, BR), :]
    ssq = jnp.sum(s_blk * s_blk, axis=1, keepdims=True)
    inv_s = 1.0 / jnp.maximum(jnp.sqrt(ssq), 1e-12)

    dots = lax.dot_general(s_blk.astype(jnp.bfloat16),
                           tp.astype(jnp.bfloat16),
                           (((1,), (1,)), ((), ())),
                           preferred_element_type=jnp.float32)
    logits = dots * inv_s * inv_t.reshape(1, NP) * (1.0 / TEMPERATURE)

    t_blk = tp_ref[pl.ds(base, BR), :]
    tsq = jnp.sum(t_blk * t_blk, axis=1, keepdims=True)
    inv_t_blk = 1.0 / jnp.maximum(jnp.sqrt(tsq), 1e-12)
    pos = (jnp.sum(s_blk * t_blk, axis=1, keepdims=True)
           * inv_s * inv_t_blk * (1.0 / TEMPERATURE))

    key_r = key_ref[0, pl.ds(base, BR)].reshape(BR, 1)
    key_c = key_ref[0, :].reshape(1, NP)
    col_ids = lax.broadcasted_iota(jnp.int32, (1, NP), 1)
    neg_mask = (key_r != key_c) & (col_ids < N_SAMPLES)

    esum = jnp.sum(jnp.where(neg_mask, jnp.exp(logits), 0.0), axis=1,
                   keepdims=True)
    log_prob = pos - jnp.log(jnp.exp(pos) + esum)

    row_ids = base + lax.broadcasted_iota(jnp.int32, (BR, 1), 0)
    w_r = wgt_ref[0, pl.ds(base, BR)].reshape(BR, 1)
    w_r = jnp.where(row_ids < N_SAMPLES, w_r, 0.0)
    contrib = jnp.sum(log_prob * w_r)

    @pl.when(r == 0)
    def _():
        out_ref[...] = jnp.zeros((1, 1), jnp.float32)
    out_ref[...] += contrib.reshape(1, 1)


def _contrastive_sum(sp, tp, keys, weights):
    return pl.pallas_call(
        _contrastive_body,
        out_shape=jax.ShapeDtypeStruct((1, 1), jnp.float32),
        grid=(NP // BR,),
        in_specs=[
            pl.BlockSpec((NP, C), lambda r: (0, 0)),
            pl.BlockSpec((NP, C), lambda r: (0, 0)),
            pl.BlockSpec((1, NP), lambda r: (0, 0)),
            pl.BlockSpec((1, NP), lambda r: (0, 0)),
        ],
        out_specs=pl.BlockSpec((1, 1), lambda r: (0, 0)),
    )(sp, tp, keys, weights)


def kernel(student_feat, teacher_feat, teacher_logits, conv1_w, conv2_w,
           conv2_b):
    mp, labf = _softmax_stage(teacher_logits)
    mp_pad = jnp.pad(mp, ((0, 0), (1, 1), (1, 1)))
    w1 = conv1_w.reshape(16, 9)
    w2 = conv2_w.reshape(1, 16)
    b2 = conv2_b.reshape(1, 1)
    raw, bmax = _conv_stage(mp_pad, w1, w2, b2)

    m = jnp.max(bmax).reshape(1)
    wraw = raw.reshape(NROW, 128)
    labf2d = labf.reshape(NROW, 128)

    g2d, key2d, wgt2d = _sampler_stage(wraw, labf2d, m)
    gidx = g2d.T.reshape(NP)
    keys = key2d.T.reshape(NP)
    weights = wgt2d.T.reshape(NP)

    cos_sum = _l2_stage(student_feat, teacher_feat)[0, 0]
    l2_loss = (2.0 / C) * (cos_sum / NPIX)

    s2d = student_feat.reshape(NVROW, 16)
    t2d = teacher_feat.reshape(NVROW, 16)
    s_pix, t_pix = _sc_gather(s2d, t2d, gidx)

    total = _contrastive_sum(s_pix, t_pix, keys.reshape(1, NP),
                             weights.reshape(1, NP))[0, 0]
    contrastive_loss = -total / N_SAMPLES

    return contrastive_loss + L2_WEIGHT * l2_loss
